# initial kernel scaffold (unmeasured)
import jax
import jax.numpy as jnp
from jax import lax
from jax.experimental import pallas as pl
from jax.experimental.pallas import tpu as pltpu

N_DEV = 4
HOPS = (2, 1, 3)


def kernel(x, w_mat):
    m_per, k = x.shape
    _, n = w_mat.shape
    n_per = n // N_DEV

    def body(x_ref, w_ref, out_ref, send_ref, recv_ref, send_sems, recv_sems):
        my = lax.axis_index("i")

        barrier_sem = pltpu.get_barrier_semaphore()
        for off in (1, 2, 3):
            pl.semaphore_signal(
                barrier_sem, inc=1,
                device_id=(lax.rem(my + off, N_DEV),),
                device_id_type=pl.DeviceIdType.MESH,
            )
        pl.semaphore_wait(barrier_sem, N_DEV - 1)

        x_bf16 = x_ref[...].astype(jnp.bfloat16)

        rdmas = {}
        for h in HOPS:
            dst = lax.rem(my + h, N_DEV)
            w_blk = w_ref[:, pl.ds(dst * n_per, n_per)].astype(jnp.bfloat16)
            y = jnp.maximum(
                jnp.dot(x_bf16, w_blk, preferred_element_type=jnp.float32), 0.0
            )
            send_ref[h - 1, :, :] = y.astype(jnp.bfloat16)
            rdma = pltpu.make_async_remote_copy(
                src_ref=send_ref.at[h - 1],
                dst_ref=recv_ref.at[h - 1],
                send_sem=send_sems.at[h - 1],
                recv_sem=recv_sems.at[h - 1],
                device_id=(dst,),
                device_id_type=pl.DeviceIdType.MESH,
            )
            rdma.start()
            rdmas[h] = rdma

        w_blk = w_ref[:, pl.ds(my * n_per, n_per)].astype(jnp.bfloat16)
        y = jnp.maximum(
            jnp.dot(x_bf16, w_blk, preferred_element_type=jnp.float32), 0.0
        )
        out_ref[pl.ds(my * m_per, m_per), :] = y

        for h in HOPS:
            src_dev = lax.rem(my + (N_DEV - h), N_DEV)
            rdmas[h].wait_recv()
            out_ref[pl.ds(src_dev * m_per, m_per), :] = (
                recv_ref[h - 1, :, :].astype(jnp.float32)
            )
        for h in HOPS:
            rdmas[h].wait_send()

    return pl.pallas_call(
        body,
        out_shape=jax.ShapeDtypeStruct((N_DEV * m_per, n_per), jnp.float32),
        in_specs=[
            pl.BlockSpec(memory_space=pltpu.VMEM),
            pl.BlockSpec(memory_space=pltpu.VMEM),
        ],
        out_specs=pl.BlockSpec(memory_space=pltpu.VMEM),
        scratch_shapes=[
            pltpu.VMEM((N_DEV - 1, m_per, n_per), jnp.bfloat16),
            pltpu.VMEM((N_DEV - 1, m_per, n_per), jnp.bfloat16),
            pltpu.SemaphoreType.DMA((N_DEV - 1,)),
            pltpu.SemaphoreType.DMA((N_DEV - 1,)),
        ],
        compiler_params=pltpu.CompilerParams(collective_id=0),
    )(x, w_mat)


# baseline (device time: 50952 ns/iter reference)
import jax
import jax.numpy as jnp
from jax import lax
from jax.experimental import pallas as pl
from jax.experimental.pallas import tpu as pltpu

N_DEV = 4
HOPS = (2, 1, 3)
X_CHUNKS = 4


def kernel(x, w_mat):
    m_per, k = x.shape
    _, n = w_mat.shape
    n_per = n // N_DEV
    m_chunk = m_per // X_CHUNKS

    def body(x_hbm, w_hbm, out_ref, x_stage, x_bf, w_stage, w_bf,
             send_ref, recv_ref, copy_sems, send_sems, recv_sems):
        my = lax.axis_index("i")

        barrier_sem = pltpu.get_barrier_semaphore()
        for off in (1, 2, 3):
            pl.semaphore_signal(
                barrier_sem, inc=1,
                device_id=(lax.rem(my + off, N_DEV),),
                device_id_type=pl.DeviceIdType.MESH,
            )
        pl.semaphore_wait(barrier_sem, N_DEV - 1)

        dsts = [lax.rem(my + h, N_DEV) for h in HOPS] + [my]

        def w_copy(idx, slot):
            return pltpu.make_async_copy(
                w_hbm.at[:, pl.ds(dsts[idx] * n_per, n_per)],
                w_stage.at[slot],
                copy_sems.at[2 + slot],
            )

        def x_copy(c, slot):
            return pltpu.make_async_copy(
                x_hbm.at[pl.ds(c * m_chunk, m_chunk), :],
                x_stage.at[slot],
                copy_sems.at[slot],
            )

        w_copy(0, 0).start()
        x_copy(0, 0).start()
        x_copy(1, 1).start()

        for c in range(X_CHUNKS):
            slot = c % 2
            x_copy(c, slot).wait()
            if c + 2 < X_CHUNKS:
                x_copy(c + 2, slot).start()
            x_bf[pl.ds(c * m_chunk, m_chunk), :] = (
                x_stage[slot].astype(jnp.bfloat16)
            )

        rdmas = {}
        for idx in range(N_DEV):
            slot = idx % 2
            w_copy(idx, slot).wait()
            if idx + 1 < N_DEV:
                w_copy(idx + 1, (idx + 1) % 2).start()
            w_bf[...] = w_stage[slot].astype(jnp.bfloat16)
            y = jnp.maximum(
                jnp.dot(x_bf[...], w_bf[...],
                        preferred_element_type=jnp.float32),
                0.0,
            )
            if idx < N_DEV - 1:
                h = HOPS[idx]
                send_ref[h - 1, :, :] = y.astype(jnp.bfloat16)
                rdma = pltpu.make_async_remote_copy(
                    src_ref=send_ref.at[h - 1],
                    dst_ref=recv_ref.at[h - 1],
                    send_sem=send_sems.at[h - 1],
                    recv_sem=recv_sems.at[h - 1],
                    device_id=(dsts[idx],),
                    device_id_type=pl.DeviceIdType.MESH,
                )
                rdma.start()
                rdmas[h] = rdma
            else:
                out_ref[pl.ds(my * m_per, m_per), :] = y

        for h in HOPS:
            src_dev = lax.rem(my + (N_DEV - h), N_DEV)
            rdmas[h].wait_recv()
            out_ref[pl.ds(src_dev * m_per, m_per), :] = (
                recv_ref[h - 1, :, :].astype(jnp.float32)
            )
        for h in HOPS:
            rdmas[h].wait_send()

    return pl.pallas_call(
        body,
        out_shape=jax.ShapeDtypeStruct((N_DEV * m_per, n_per), jnp.float32),
        in_specs=[
            pl.BlockSpec(memory_space=pl.ANY),
            pl.BlockSpec(memory_space=pl.ANY),
        ],
        out_specs=pl.BlockSpec(memory_space=pltpu.VMEM),
        scratch_shapes=[
            pltpu.VMEM((2, m_chunk, k), jnp.float32),
            pltpu.VMEM((m_per, k), jnp.bfloat16),
            pltpu.VMEM((2, k, n_per), jnp.float32),
            pltpu.VMEM((k, n_per), jnp.bfloat16),
            pltpu.VMEM((N_DEV - 1, m_per, n_per), jnp.bfloat16),
            pltpu.VMEM((N_DEV - 1, m_per, n_per), jnp.bfloat16),
            pltpu.SemaphoreType.DMA((4,)),
            pltpu.SemaphoreType.DMA((N_DEV - 1,)),
            pltpu.SemaphoreType.DMA((N_DEV - 1,)),
        ],
        compiler_params=pltpu.CompilerParams(
            collective_id=0,
            vmem_limit_bytes=100 * 1024 * 1024,
        ),
    )(x, w_mat)


# device time: 44664 ns/iter; 1.1408x vs baseline; 1.1408x over previous
import jax
import jax.numpy as jnp
from jax import lax
from jax.experimental import pallas as pl
from jax.experimental.pallas import tpu as pltpu

N_DEV = 4
HOPS = (2, 1, 3)
X_CHUNKS = 4
NPIECE = 4


def kernel(x, w_mat):
    m_per, k = x.shape
    _, n = w_mat.shape
    n_per = n // N_DEV
    m_chunk = m_per // X_CHUNKS

    def body(x_hbm, w_hbm, out_ref, x_stage, x_bf, w_stage, w_bf,
             send_ref, recv_ref, copy_sems, send_sems, recv_sems):
        my = lax.axis_index("i")

        dsts = [lax.rem(my + h, N_DEV) for h in HOPS] + [my]

        def w_copy(idx, slot):
            return pltpu.make_async_copy(
                w_hbm.at[:, pl.ds(dsts[idx] * n_per, n_per)],
                w_stage.at[slot],
                copy_sems.at[2 + slot],
            )

        def x_copy(c, slot):
            return pltpu.make_async_copy(
                x_hbm.at[pl.ds(c * m_chunk, m_chunk), :],
                x_stage.at[slot],
                copy_sems.at[slot],
            )

        w_copy(0, 0).start()
        x_copy(0, 0).start()
        x_copy(1, 1).start()

        barrier_sem = pltpu.get_barrier_semaphore()
        for off in (1, 2, 3):
            pl.semaphore_signal(
                barrier_sem, inc=1,
                device_id=(lax.rem(my + off, N_DEV),),
                device_id_type=pl.DeviceIdType.MESH,
            )
        pl.semaphore_wait(barrier_sem, N_DEV - 1)

        rdmas = {}

        def emit_piece(idx, c):
            rows = pl.ds(c * m_chunk, m_chunk)
            wslot = idx % 2
            y = jnp.maximum(
                jnp.dot(x_bf[rows, :], w_bf[wslot],
                        preferred_element_type=jnp.float32),
                0.0,
            )
            if idx < N_DEV - 1:
                h = HOPS[idx]
                send_ref[h - 1, rows, :] = y.astype(jnp.bfloat16)
                rdma = pltpu.make_async_remote_copy(
                    src_ref=send_ref.at[h - 1, rows, :],
                    dst_ref=recv_ref.at[h - 1, rows, :],
                    send_sem=send_sems.at[h - 1, c],
                    recv_sem=recv_sems.at[h - 1, c],
                    device_id=(dsts[idx],),
                    device_id_type=pl.DeviceIdType.MESH,
                )
                rdma.start()
                rdmas[(h, c)] = rdma
            else:
                out_ref[pl.ds(my * m_per + c * m_chunk, m_chunk), :] = y

        for c in range(X_CHUNKS):
            slot = c % 2
            x_copy(c, slot).wait()
            if c + 2 < X_CHUNKS:
                x_copy(c + 2, slot).start()
            x_bf[pl.ds(c * m_chunk, m_chunk), :] = (
                x_stage[slot].astype(jnp.bfloat16)
            )
            if c == 0:
                w_copy(0, 0).wait()
                w_copy(1, 1).start()
                w_bf[0] = w_stage[0].astype(jnp.bfloat16)
            emit_piece(0, c)

        for idx in range(1, N_DEV):
            wslot = idx % 2
            w_copy(idx, wslot).wait()
            if idx + 1 < N_DEV:
                w_copy(idx + 1, (idx + 1) % 2).start()
            w_bf[wslot] = w_stage[wslot].astype(jnp.bfloat16)
            for c in range(NPIECE):
                emit_piece(idx, c)

        for i, h in enumerate(HOPS):
            src_dev = lax.rem(my + (N_DEV - h), N_DEV)
            for c in range(NPIECE):
                rdmas[(h, c)].wait_recv()
                out_ref[pl.ds(src_dev * m_per + c * m_chunk, m_chunk), :] = (
                    recv_ref[h - 1, pl.ds(c * m_chunk, m_chunk), :]
                    .astype(jnp.float32)
                )
        for key in rdmas:
            rdmas[key].wait_send()

    return pl.pallas_call(
        body,
        out_shape=jax.ShapeDtypeStruct((N_DEV * m_per, n_per), jnp.float32),
        in_specs=[
            pl.BlockSpec(memory_space=pl.ANY),
            pl.BlockSpec(memory_space=pl.ANY),
        ],
        out_specs=pl.BlockSpec(memory_space=pltpu.VMEM),
        scratch_shapes=[
            pltpu.VMEM((2, m_chunk, k), jnp.float32),
            pltpu.VMEM((m_per, k), jnp.bfloat16),
            pltpu.VMEM((2, k, n_per), jnp.float32),
            pltpu.VMEM((2, k, n_per), jnp.bfloat16),
            pltpu.VMEM((N_DEV - 1, m_per, n_per), jnp.bfloat16),
            pltpu.VMEM((N_DEV - 1, m_per, n_per), jnp.bfloat16),
            pltpu.SemaphoreType.DMA((4,)),
            pltpu.SemaphoreType.DMA((N_DEV - 1, NPIECE)),
            pltpu.SemaphoreType.DMA((N_DEV - 1, NPIECE)),
        ],
        compiler_params=pltpu.CompilerParams(
            collective_id=0,
            vmem_limit_bytes=100 * 1024 * 1024,
        ),
    )(x, w_mat)


# device time: 44324 ns/iter; 1.1495x vs baseline; 1.0077x over previous
import jax
import jax.numpy as jnp
from jax import lax
from jax.experimental import pallas as pl
from jax.experimental.pallas import tpu as pltpu

N_DEV = 4
X_CHUNKS = 4
NPIECE = 4


def kernel(x, w_mat):
    m_per, k = x.shape
    _, n = w_mat.shape
    n_per = n // N_DEV
    m_chunk = m_per // X_CHUNKS

    def body(x_hbm, w_hbm, out_ref, x_stage, x_bf, w_stage, w_bf,
             send_ref, recv_ref, copy_sems, send_sems, recv_sems):
        my = lax.axis_index("i")

        def w_copy(h, stage_slot):
            dst = lax.rem(my + h, N_DEV)
            return pltpu.make_async_copy(
                w_hbm.at[:, pl.ds(dst * n_per, n_per)],
                w_stage.at[stage_slot],
                copy_sems.at[2 + stage_slot],
            )

        def x_copy(c, slot):
            return pltpu.make_async_copy(
                x_hbm.at[pl.ds(c * m_chunk, m_chunk), :],
                x_stage.at[slot],
                copy_sems.at[slot],
            )

        w_copy(2, 0).start()
        x_copy(0, 0).start()

        barrier_sem = pltpu.get_barrier_semaphore()
        for off in (1, 2, 3):
            pl.semaphore_signal(
                barrier_sem, inc=1,
                device_id=(lax.rem(my + off, N_DEV),),
                device_id_type=pl.DeviceIdType.MESH,
            )
        pl.semaphore_wait(barrier_sem, N_DEV - 1)

        x_copy(1, 1).start()

        rdmas = {}

        def emit_piece(h, c, wslot):
            rows = pl.ds(c * m_chunk, m_chunk)
            y = jnp.maximum(
                jnp.dot(x_bf[rows, :], w_bf[wslot],
                        preferred_element_type=jnp.float32),
                0.0,
            )
            if h:
                send_ref[h - 1, rows, :] = y.astype(jnp.bfloat16)
                rdma = pltpu.make_async_remote_copy(
                    src_ref=send_ref.at[h - 1, rows, :],
                    dst_ref=recv_ref.at[h - 1, rows, :],
                    send_sem=send_sems.at[h - 1, c],
                    recv_sem=recv_sems.at[h - 1, c],
                    device_id=(lax.rem(my + h, N_DEV),),
                    device_id_type=pl.DeviceIdType.MESH,
                )
                rdma.start()
                rdmas[(h, c)] = rdma
            else:
                out_ref[pl.ds(my * m_per + c * m_chunk, m_chunk), :] = y

        for c in range(X_CHUNKS):
            slot = c % 2
            x_copy(c, slot).wait()
            if c + 2 < X_CHUNKS:
                x_copy(c + 2, slot).start()
            x_bf[pl.ds(c * m_chunk, m_chunk), :] = (
                x_stage[slot].astype(jnp.bfloat16)
            )
            if c == 0:
                w_copy(2, 0).wait()
                w_copy(3, 1).start()
                w_bf[0] = w_stage[0].astype(jnp.bfloat16)
                w_copy(1, 0).start()
            emit_piece(2, c, 0)

        w_copy(3, 1).wait()
        w_bf[1] = w_stage[1].astype(jnp.bfloat16)
        w_copy(0, 1).start()
        w_copy(1, 0).wait()
        w_bf[2] = w_stage[0].astype(jnp.bfloat16)
        for c in range(NPIECE):
            emit_piece(3, c, 1)
            emit_piece(1, c, 2)

        w_copy(0, 1).wait()
        w_bf[0] = w_stage[1].astype(jnp.bfloat16)
        for c in range(NPIECE):
            emit_piece(0, c, 0)

        def drain(h, c):
            src_dev = lax.rem(my + (N_DEV - h), N_DEV)
            rdmas[(h, c)].wait_recv()
            out_ref[pl.ds(src_dev * m_per + c * m_chunk, m_chunk), :] = (
                recv_ref[h - 1, pl.ds(c * m_chunk, m_chunk), :]
                .astype(jnp.float32)
            )

        for c in range(NPIECE):
            drain(2, c)
        for c in range(NPIECE):
            drain(3, c)
            drain(1, c)
        for key in rdmas:
            rdmas[key].wait_send()

    return pl.pallas_call(
        body,
        out_shape=jax.ShapeDtypeStruct((N_DEV * m_per, n_per), jnp.float32),
        in_specs=[
            pl.BlockSpec(memory_space=pl.ANY),
            pl.BlockSpec(memory_space=pl.ANY),
        ],
        out_specs=pl.BlockSpec(memory_space=pltpu.VMEM),
        scratch_shapes=[
            pltpu.VMEM((2, m_chunk, k), jnp.float32),
            pltpu.VMEM((m_per, k), jnp.bfloat16),
            pltpu.VMEM((2, k, n_per), jnp.float32),
            pltpu.VMEM((3, k, n_per), jnp.bfloat16),
            pltpu.VMEM((N_DEV - 1, m_per, n_per), jnp.bfloat16),
            pltpu.VMEM((N_DEV - 1, m_per, n_per), jnp.bfloat16),
            pltpu.SemaphoreType.DMA((4,)),
            pltpu.SemaphoreType.DMA((N_DEV - 1, NPIECE)),
            pltpu.SemaphoreType.DMA((N_DEV - 1, NPIECE)),
        ],
        compiler_params=pltpu.CompilerParams(
            collective_id=0,
            vmem_limit_bytes=100 * 1024 * 1024,
        ),
    )(x, w_mat)


# device time: 42154 ns/iter; 1.2087x vs baseline; 1.0515x over previous
import jax
import jax.numpy as jnp
from jax import lax
from jax.experimental import pallas as pl
from jax.experimental.pallas import tpu as pltpu

N_DEV = 4
X_CHUNKS = 4
NPIECE = 4


def kernel(x, w_mat):
    m_per, k = x.shape
    _, n = w_mat.shape
    n_per = n // N_DEV
    m_chunk = m_per // X_CHUNKS

    def body(x_hbm, w_hbm, out_hbm, x_stage, x_bf, w_stage, w_bf,
             send_ref, recv_ref, ostage, copy_sems, send_sems, recv_sems,
             out_sems):
        my = lax.axis_index("i")

        def out_dma(row_start, nrows, hop_idx, c):
            return pltpu.make_async_copy(
                ostage.at[pl.ds(row_start, nrows), :],
                out_hbm.at[pl.ds(row_start, nrows), :],
                out_sems.at[hop_idx, c],
            )

        def w_copy(h, stage_slot):
            dst = lax.rem(my + h, N_DEV)
            return pltpu.make_async_copy(
                w_hbm.at[:, pl.ds(dst * n_per, n_per)],
                w_stage.at[stage_slot],
                copy_sems.at[2 + stage_slot],
            )

        def x_copy(c, slot):
            return pltpu.make_async_copy(
                x_hbm.at[pl.ds(c * m_chunk, m_chunk), :],
                x_stage.at[slot],
                copy_sems.at[slot],
            )

        w_copy(2, 0).start()
        x_copy(0, 0).start()

        barrier_sem = pltpu.get_barrier_semaphore()
        for off in (1, 2, 3):
            pl.semaphore_signal(
                barrier_sem, inc=1,
                device_id=(lax.rem(my + off, N_DEV),),
                device_id_type=pl.DeviceIdType.MESH,
            )
        pl.semaphore_wait(barrier_sem, N_DEV - 1)

        x_copy(1, 1).start()

        rdmas = {}

        def emit_piece(h, c, wslot):
            rows = pl.ds(c * m_chunk, m_chunk)
            y = jnp.maximum(
                jnp.dot(x_bf[rows, :], w_bf[wslot],
                        preferred_element_type=jnp.float32),
                0.0,
            )
            if h:
                send_ref[h - 1, rows, :] = y.astype(jnp.bfloat16)
                rdma = pltpu.make_async_remote_copy(
                    src_ref=send_ref.at[h - 1, rows, :],
                    dst_ref=recv_ref.at[h - 1, rows, :],
                    send_sem=send_sems.at[h - 1, c],
                    recv_sem=recv_sems.at[h - 1, c],
                    device_id=(lax.rem(my + h, N_DEV),),
                    device_id_type=pl.DeviceIdType.MESH,
                )
                rdma.start()
                rdmas[(h, c)] = rdma
            else:
                row = my * m_per + c * m_chunk
                ostage[pl.ds(row, m_chunk), :] = y
                out_dma(row, m_chunk, 3, c).start()

        for c in range(X_CHUNKS):
            slot = c % 2
            x_copy(c, slot).wait()
            if c + 2 < X_CHUNKS:
                x_copy(c + 2, slot).start()
            x_bf[pl.ds(c * m_chunk, m_chunk), :] = (
                x_stage[slot].astype(jnp.bfloat16)
            )
            if c == 0:
                w_copy(2, 0).wait()
                w_copy(3, 1).start()
                w_bf[0] = w_stage[0].astype(jnp.bfloat16)
                w_copy(1, 0).start()
            emit_piece(2, c, 0)

        w_copy(3, 1).wait()
        w_bf[1] = w_stage[1].astype(jnp.bfloat16)
        w_copy(0, 1).start()
        w_copy(1, 0).wait()
        w_bf[2] = w_stage[0].astype(jnp.bfloat16)
        for c in range(NPIECE):
            emit_piece(3, c, 1)
            emit_piece(1, c, 2)

        w_copy(0, 1).wait()
        w_bf[0] = w_stage[1].astype(jnp.bfloat16)
        for c in range(NPIECE):
            emit_piece(0, c, 0)

        def drain(h, c):
            src_dev = lax.rem(my + (N_DEV - h), N_DEV)
            rdmas[(h, c)].wait_recv()
            row = src_dev * m_per + c * m_chunk
            ostage[pl.ds(row, m_chunk), :] = (
                recv_ref[h - 1, pl.ds(c * m_chunk, m_chunk), :]
                .astype(jnp.float32)
            )
            out_dma(row, m_chunk, h - 1, c).start()

        for c in range(NPIECE):
            drain(2, c)
        for c in range(NPIECE):
            drain(3, c)
            drain(1, c)
        for key in rdmas:
            rdmas[key].wait_send()
        for hop_idx, h in ((0, 2), (1, 3), (2, 1), (3, 0)):
            for c in range(NPIECE):
                src_dev = lax.rem(my + (N_DEV - h), N_DEV)
                out_dma(src_dev * m_per + c * m_chunk, m_chunk,
                        hop_idx, c).wait()

    return pl.pallas_call(
        body,
        out_shape=jax.ShapeDtypeStruct((N_DEV * m_per, n_per), jnp.float32),
        in_specs=[
            pl.BlockSpec(memory_space=pl.ANY),
            pl.BlockSpec(memory_space=pl.ANY),
        ],
        out_specs=pl.BlockSpec(memory_space=pl.ANY),
        scratch_shapes=[
            pltpu.VMEM((2, m_chunk, k), jnp.float32),
            pltpu.VMEM((m_per, k), jnp.bfloat16),
            pltpu.VMEM((2, k, n_per), jnp.float32),
            pltpu.VMEM((3, k, n_per), jnp.bfloat16),
            pltpu.VMEM((N_DEV - 1, m_per, n_per), jnp.bfloat16),
            pltpu.VMEM((N_DEV - 1, m_per, n_per), jnp.bfloat16),
            pltpu.VMEM((N_DEV * m_per, n_per), jnp.float32),
            pltpu.SemaphoreType.DMA((4,)),
            pltpu.SemaphoreType.DMA((N_DEV - 1, NPIECE)),
            pltpu.SemaphoreType.DMA((N_DEV - 1, NPIECE)),
            pltpu.SemaphoreType.DMA((N_DEV, NPIECE)),
        ],
        compiler_params=pltpu.CompilerParams(
            collective_id=0,
            vmem_limit_bytes=100 * 1024 * 1024,
        ),
    )(x, w_mat)
